# baseline (speedup 1.0000x reference)
import functools

import jax
import jax.numpy as jnp
from jax import lax
from jax.experimental import pallas as pl
from jax.experimental.pallas import tpu as pltpu
from jax.experimental.pallas import tpu_sc as plsc

B = 16384
DU, DI, DC, DN = 64, 64, 16, 13

NC, NS = 2, 16
NW = NC * NS
BPW = B // NW


def _gather_body(uid, iid, cid, ut, it, ct, ue_o, ie_o, ce_o,
                 idx_u, idx_i, idx_c, rows_u, rows_i, rows_c, sem):
    wid = lax.axis_index("s") * NC + lax.axis_index("c")
    base = wid * BPW
    pltpu.sync_copy(uid.at[pl.ds(base, BPW)], idx_u)
    pltpu.sync_copy(iid.at[pl.ds(base, BPW)], idx_i)
    pltpu.sync_copy(cid.at[pl.ds(base, BPW)], idx_c)
    cu = pltpu.async_copy(ut.at[idx_u], rows_u, sem)
    ci = pltpu.async_copy(it.at[idx_i], rows_i, sem)
    cc = pltpu.async_copy(ct.at[idx_c], rows_c, sem)
    cu.wait()
    pltpu.sync_copy(rows_u, ue_o.at[pl.ds(base, BPW)])
    ci.wait()
    pltpu.sync_copy(rows_i, ie_o.at[pl.ds(base, BPW)])
    cc.wait()
    pltpu.sync_copy(rows_c, ce_o.at[pl.ds(base, BPW)])


def _sc_gather(uid, iid, cid, ut, it, ct):
    mesh = plsc.VectorSubcoreMesh(core_axis_name="c", subcore_axis_name="s")
    f = functools.partial(
        pl.kernel,
        mesh=mesh,
        out_type=[
            jax.ShapeDtypeStruct((B, DU), jnp.float32),
            jax.ShapeDtypeStruct((B, DI), jnp.float32),
            jax.ShapeDtypeStruct((B, DC), jnp.float32),
        ],
        scratch_types=[
            pltpu.VMEM((BPW,), jnp.int32),
            pltpu.VMEM((BPW,), jnp.int32),
            pltpu.VMEM((BPW,), jnp.int32),
            pltpu.VMEM((BPW, DU), jnp.float32),
            pltpu.VMEM((BPW, DI), jnp.float32),
            pltpu.VMEM((BPW, DC), jnp.float32),
            pltpu.SemaphoreType.DMA,
        ],
        compiler_params=pltpu.CompilerParams(use_tc_tiling_on_sc=False),
    )(_gather_body)
    return f(uid, iid, cid, ut, it, ct)


def _mlp_body(ue, ie, ce, nf, w1u, w1i, w1c, w1n, b1, g1, be1,
              w2, b2, g2, be2, w3, b3, out):
    h = jnp.dot(ue[...], w1u[...], preferred_element_type=jnp.float32)
    h = h + jnp.dot(ie[...], w1i[...], preferred_element_type=jnp.float32)
    h = h + jnp.dot(ce[...], w1c[...], preferred_element_type=jnp.float32)
    h = h + jnp.dot(nf[...], w1n[...], preferred_element_type=jnp.float32)
    h = h + b1[...]
    m = jnp.mean(h, axis=0, keepdims=True)
    d = h - m
    v = jnp.mean(d * d, axis=0, keepdims=True)
    h = jnp.maximum(g1[...] * d * lax.rsqrt(v + 1e-5) + be1[...], 0.0)

    h = jnp.dot(h, w2[...], preferred_element_type=jnp.float32) + b2[...]
    m = jnp.mean(h, axis=0, keepdims=True)
    d = h - m
    v = jnp.mean(d * d, axis=0, keepdims=True)
    h = jnp.maximum(g2[...] * d * lax.rsqrt(v + 1e-5) + be2[...], 0.0)

    o = jnp.dot(h, w3[...], preferred_element_type=jnp.float32) + b3[...]
    out[...] = 1.0 / (1.0 + jnp.exp(-o))


def _mlp(ue, ie, ce, nf, W1, b1, g1, be1, W2, b2, g2, be2, W3, b3):
    W1T = W1.T
    args = (
        ue, ie, ce, nf,
        W1T[0:DU], W1T[DU:DU + DI], W1T[DU + DI:DU + DI + DC],
        W1T[DU + DI + DC:], b1.reshape(1, -1), g1.reshape(1, -1),
        be1.reshape(1, -1), W2.T, b2.reshape(1, -1), g2.reshape(1, -1),
        be2.reshape(1, -1), W3.T, b3.reshape(1, 1),
    )
    return pl.pallas_call(
        _mlp_body,
        out_shape=jax.ShapeDtypeStruct((B, 1), jnp.float32),
    )(*args)


def kernel(user_ids, item_ids, category_ids, numerical_features,
           user_table, item_table, cat_table,
           W1, b1, g1, be1, W2, b2, g2, be2, W3, b3):
    ue, ie, ce = _sc_gather(user_ids, item_ids, category_ids,
                            user_table, item_table, cat_table)
    return _mlp(ue, ie, ce, numerical_features,
                W1, b1, g1, be1, W2, b2, g2, be2, W3, b3)
